# baseline (speedup 1.0000x reference)
import functools

import jax
import jax.numpy as jnp
from jax import lax
from jax.experimental import pallas as pl
from jax.experimental.pallas import tpu as pltpu
from jax.experimental.pallas import tpu_sc as plsc

N_NODES = 10000
N_EDGES = 320000
HIDDEN = 128
NUM_BESSEL = 8
NUM_GRAPHS = 8
R_MAX = 5.0
R2_CUT = R_MAX * R_MAX
L2_SENTINEL = 4.0 * R2_CUT
P_CUTOFF = 5
AVG_NUM_NEIGHBORS = 32.0

NC = 2
NS = 16
NW = NC * NS
E_PER_W = N_EDGES // NW
CHUNK = 80
N_CHUNKS = E_PER_W // CHUNK
SEG = 10240
N_PAD = 10240
ROWS_PER_S = N_PAD // NS
EDGE_BLK = 2560
N_EDGE_BLKS = NW * SEG // EDGE_BLK
LAG = 8


def _k1_body(px_hbm, py_hbm, pz_hbm, src_hbm, dst_hbm,
             l2c_hbm, srcc_hbm, dstc_hbm, counts_hbm,
             sidx_v, didx_v, gbuf, sent_l2, sent_i, idxbuf, stage_l2, cbuf,
             sem, sem2):
  wid = lax.axis_index("c") * NS + lax.axis_index("s")
  base_o = wid * SEG

  pltpu.sync_copy(src_hbm.at[wid], sidx_v)
  pltpu.sync_copy(dst_hbm.at[wid], didx_v)

  def sfill(i, _):
    sl = pl.ds(i * 16, 16)
    sent_l2[sl] = jnp.full((16,), L2_SENTINEL, jnp.float32)
    sent_i[sl] = jnp.zeros((16,), jnp.int32)
    return 0
  lax.fori_loop(0, SEG // 16, sfill, 0)
  pltpu.sync_copy(sent_l2, l2c_hbm.at[pl.ds(base_o, SEG)])
  pltpu.sync_copy(sent_i, srcc_hbm.at[pl.ds(base_o, SEG)])
  pltpu.sync_copy(sent_i, dstc_hbm.at[pl.ds(base_o, SEG)])

  tabs = (px_hbm, py_hbm, pz_hbm)

  def fire_g(k, b):
    for c in range(3):
      pltpu.async_copy(tabs[c].at[sidx_v.at[k]], gbuf.at[b, c], sem)
      pltpu.async_copy(tabs[c].at[didx_v.at[k]], gbuf.at[b, 3 + c], sem)

  def drain_g(k, b):
    for c in range(3):
      pltpu.make_async_copy(tabs[c].at[sidx_v.at[k]], gbuf.at[b, c],
                            sem).wait()
      pltpu.make_async_copy(tabs[c].at[didx_v.at[k]], gbuf.at[b, 3 + c],
                            sem).wait()

  def fire_s(k):
    pltpu.async_copy(stage_l2.at[k], l2c_hbm.at[idxbuf.at[k]], sem2)
    pltpu.async_copy(sidx_v.at[k], srcc_hbm.at[idxbuf.at[k]], sem2)
    pltpu.async_copy(didx_v.at[k], dstc_hbm.at[idxbuf.at[k]], sem2)

  def drain_s(k):
    pltpu.make_async_copy(stage_l2.at[k], l2c_hbm.at[idxbuf.at[k]],
                          sem2).wait()
    pltpu.make_async_copy(sidx_v.at[k], srcc_hbm.at[idxbuf.at[k]],
                          sem2).wait()
    pltpu.make_async_copy(didx_v.at[k], dstc_hbm.at[idxbuf.at[k]],
                          sem2).wait()

  def process(k, b, cnt):
    iota = lax.iota(jnp.int32, 16)
    for j in range(CHUNK // 16):
      sl = pl.ds(j * 16, 16)
      dx = gbuf[b, 3, sl] - gbuf[b, 0, sl]
      dy = gbuf[b, 4, sl] - gbuf[b, 1, sl]
      dz = gbuf[b, 5, sl] - gbuf[b, 2, sl]
      l2v = dx * dx + dy * dy + dz * dz
      mask = l2v < R2_CUT
      cs = jnp.where(mask, jnp.full((16,), 1, jnp.int32),
                     jnp.zeros((16,), jnp.int32))
      for d in (1, 2, 4, 8):
        sh = jnp.take(cs, jnp.maximum(iota - d, 0))
        cs = cs + jnp.where(iota >= d, sh, 0)
      idx = jnp.where(mask, base_o + cnt + cs - 1,
                      base_o + E_PER_W + j * 16 + iota)
      idxbuf[k, sl] = idx
      stage_l2[k, sl] = l2v
      cnt = cnt + cs[15]
    fire_s(k)

    @pl.when(k >= LAG)
    def _():
      drain_s(k - LAG)
    return cnt

  def chunk_body(k, cnt):
    cps = []
    for c in range(3):
      cps.append(pltpu.async_copy(tabs[c].at[sidx_v.at[k]],
                                  gbuf.at[0, c], sem))
      cps.append(pltpu.async_copy(tabs[c].at[didx_v.at[k]],
                                  gbuf.at[0, 3 + c], sem))
    for cp in cps:
      cp.wait()
    cnt = process(k, 0, cnt)
    return cnt

  cnt = lax.fori_loop(0, N_CHUNKS, chunk_body, jnp.int32(0))

  def tail_drain(k, _):
    drain_s(k)
    return 0
  lax.fori_loop(N_CHUNKS - LAG, N_CHUNKS, tail_drain, 0)

  cbuf[...] = jnp.zeros((16,), jnp.int32) + cnt
  pltpu.sync_copy(cbuf, counts_hbm.at[wid])


_k1 = functools.partial(
    pl.kernel,
    out_type=(jax.ShapeDtypeStruct((NW * SEG,), jnp.float32),
              jax.ShapeDtypeStruct((NW * SEG,), jnp.int32),
              jax.ShapeDtypeStruct((NW * SEG,), jnp.int32),
              jax.ShapeDtypeStruct((NW, 16), jnp.int32)),
    mesh=plsc.VectorSubcoreMesh(core_axis_name="c", subcore_axis_name="s"),
    scratch_types=[
        pltpu.VMEM((N_CHUNKS, CHUNK), jnp.int32),
        pltpu.VMEM((N_CHUNKS, CHUNK), jnp.int32),
        pltpu.VMEM((2, 6, CHUNK), jnp.float32),
        pltpu.VMEM((SEG,), jnp.float32),
        pltpu.VMEM((SEG,), jnp.int32),
        pltpu.VMEM((N_CHUNKS, CHUNK), jnp.int32),
        pltpu.VMEM((N_CHUNKS, CHUNK), jnp.float32),
        pltpu.VMEM((16,), jnp.int32),
        pltpu.SemaphoreType.DMA,
        pltpu.SemaphoreType.DMA,
    ],
)(_k1_body)


def _k3_body(nf_hbm, tpw_hbm, srcc_hbm, dstc_hbm, counts_hbm, out_hbm,
             cidx_s, cidx_d, frows_v, tpw_v, cbuf, accum, sem):
  cid = lax.axis_index("c")
  sid = lax.axis_index("s")
  wid = cid * NS + sid
  base = wid * SEG

  pltpu.sync_copy(counts_hbm.at[wid], cbuf)
  cnt = cbuf[...][0]
  nch = (cnt + (CHUNK - 1)) // CHUNK

  def zrow(r, _):
    for cb in range(HIDDEN // 16):
      frows_v[0, r, pl.ds(cb * 16, 16)] = jnp.zeros((16,), jnp.float32)
    return 0
  lax.fori_loop(0, CHUNK, zrow, 0)
  for j in range(ROWS_PER_S // CHUNK):
    pltpu.sync_copy(frows_v.at[0],
                    accum.at[pl.ds(sid * ROWS_PER_S + j * CHUNK, CHUNK)])
  plsc.subcore_barrier()

  def fire(k, b):
    pltpu.sync_copy(srcc_hbm.at[wid, k], cidx_s.at[b])
    pltpu.sync_copy(dstc_hbm.at[wid, k], cidx_d.at[b])
    pltpu.async_copy(nf_hbm.at[cidx_s.at[b]], frows_v.at[b], sem)
    pltpu.async_copy(tpw_hbm.at[pl.ds(base + k * CHUNK, CHUNK)],
                     tpw_v.at[b], sem)

  def drain(k, b):
    pltpu.make_async_copy(nf_hbm.at[cidx_s.at[b]], frows_v.at[b],
                          sem).wait()
    pltpu.make_async_copy(tpw_hbm.at[pl.ds(base + k * CHUNK, CHUNK)],
                          tpw_v.at[b], sem).wait()

  def process(k, b):
    def mrow(r, _):
      for cb in range(HIDDEN // 16):
        sl = pl.ds(cb * 16, 16)
        frows_v[b, r, sl] = frows_v[b, r, sl] * tpw_v[b, r, sl]
      return 0
    lax.fori_loop(0, CHUNK, mrow, 0)
    pltpu.sync_copy(frows_v.at[b], accum.at[cidx_d.at[b]], add=True)

  @pl.when(nch > 0)
  def _():
    fire(0, 0)

  def pair_body(i, _):
    k0 = 2 * i

    @pl.when(k0 + 1 < nch)
    def _():
      fire(k0 + 1, 1)
    drain(k0, 0)
    process(k0, 0)

    @pl.when(k0 + 2 < nch)
    def _():
      fire(k0 + 2, 0)

    @pl.when(k0 + 1 < nch)
    def _():
      drain(k0 + 1, 1)
      process(k0 + 1, 1)
    return 0

  lax.fori_loop(0, (nch + 1) // 2, pair_body, 0)
  plsc.subcore_barrier()
  pltpu.sync_copy(accum.at[pl.ds(sid * ROWS_PER_S, ROWS_PER_S)],
                  out_hbm.at[cid, pl.ds(sid * ROWS_PER_S, ROWS_PER_S)])


_k3 = functools.partial(
    pl.kernel,
    out_type=jax.ShapeDtypeStruct((NC, N_PAD, HIDDEN), jnp.float32),
    mesh=plsc.VectorSubcoreMesh(core_axis_name="c", subcore_axis_name="s"),
    scratch_types=[
        pltpu.VMEM((2, CHUNK), jnp.int32),
        pltpu.VMEM((2, CHUNK), jnp.int32),
        pltpu.VMEM((2, CHUNK, HIDDEN), jnp.float32),
        pltpu.VMEM((2, CHUNK, HIDDEN), jnp.float32),
        pltpu.VMEM((16,), jnp.int32),
        pltpu.VMEM_SHARED((N_PAD, HIDDEN), jnp.float32),
        pltpu.SemaphoreType.DMA,
    ],
)(_k3_body)


def _k2a_body(na_ref, we_ref, out_ref):
  out_ref[...] = jnp.dot(na_ref[...], we_ref[...],
                         preferred_element_type=jnp.float32)


def _node_feats(node_attrs, w_embed):
  return pl.pallas_call(
      _k2a_body,
      out_shape=jax.ShapeDtypeStruct((N_NODES, HIDDEN), jnp.float32),
  )(node_attrs, w_embed)


def _k2b_body(l2_ref, wr_ref, out_ref):
  l2 = l2_ref[0, 0, :]
  lengths = jnp.sqrt(l2)
  r = jnp.maximum(lengths, 1e-6)
  n = (lax.broadcasted_iota(jnp.int32, (NUM_BESSEL, 1), 0) + 1
       ).astype(jnp.float32)
  bessel = (jnp.sqrt(2.0 / R_MAX)
            * jnp.sin(n * (jnp.pi / R_MAX) * r[None, :]) / r[None, :])
  x = lengths / R_MAX
  p = float(P_CUTOFF)
  xp = x ** p
  env = (1.0
         - ((p + 1.0) * (p + 2.0) / 2.0) * xp
         + p * (p + 2.0) * xp * x
         - (p * (p + 1.0) / 2.0) * xp * x * x)
  env = env * (x < 1.0).astype(jnp.float32)
  ef = bessel * env[None, :]
  out_ref[...] = lax.dot_general(
      ef, wr_ref[...],
      dimension_numbers=(((0,), (0,)), ((), ())),
      preferred_element_type=jnp.float32)


def _tp_w(l2, w_radial):
  l2_3d = l2.reshape(N_EDGE_BLKS, 1, EDGE_BLK)
  return pl.pallas_call(
      _k2b_body,
      grid=(N_EDGE_BLKS,),
      in_specs=[
          pl.BlockSpec((1, 1, EDGE_BLK), lambda i: (i, 0, 0)),
          pl.BlockSpec((NUM_BESSEL, HIDDEN), lambda i: (0, 0)),
      ],
      out_specs=pl.BlockSpec((EDGE_BLK, HIDDEN), lambda i: (i, 0)),
      out_shape=jax.ShapeDtypeStruct((NW * SEG, HIDDEN), jnp.float32),
  )(l2_3d, w_radial)


def _k4_body(aggp_ref, nf_ref, na_ref, batch_ref, pos_ref, mmp_ref, mmc_ref,
             aew_ref, wf_ref, wro_ref, out_ref):
  agg0 = (aggp_ref[0] + aggp_ref[1]) * (1.0 / AVG_NUM_NEIGHBORS)
  nf = nf_ref[...]
  dipole = lax.dot_general(mmc_ref[...], mmp_ref[...],
                           dimension_numbers=(((0,), (0,)), ((), ())),
                           preferred_element_type=jnp.float32)
  field_scal = lax.dot_general(pos_ref[...], dipole,
                               dimension_numbers=(((1,), (1,)), ((), ())),
                               preferred_element_type=jnp.float32)
  h = agg0 + nf + field_scal * wf_ref[...]
  h = h * jax.nn.sigmoid(h)
  ne = jnp.dot(h, wro_ref[...], preferred_element_type=jnp.float32)
  ne0 = jnp.dot(na_ref[...], aew_ref[...],
                preferred_element_type=jnp.float32)
  cat = jnp.concatenate([ne, ne0], axis=1)
  gids = lax.broadcasted_iota(jnp.int32, (N_NODES, NUM_GRAPHS), 1)
  m = (batch_ref[...] == gids).astype(jnp.float32)
  eng = lax.dot_general(m, cat,
                        dimension_numbers=(((0,), (0,)), ((), ())),
                        preferred_element_type=jnp.float32)
  out_ref[...] = eng[:, :3] + eng[:, 3:4]


def _epilogue(aggp, nf, node_attrs, batch2d, positions, mm_positions,
              mmc2d, aew2d, wf2d, w_readout):
  return pl.pallas_call(
      _k4_body,
      out_shape=jax.ShapeDtypeStruct((NUM_GRAPHS, 3), jnp.float32),
  )(aggp, nf, node_attrs, batch2d, positions, mm_positions, mmc2d,
    aew2d, wf2d, w_readout)


def kernel(positions, node_attrs, edge_index, shifts, batch, ptr,
           mm_positions, mm_charges, atomic_energies_w, W_embed,
           W_radial, W_field, W_readout):
  del ptr
  del shifts
  src3 = edge_index[0].astype(jnp.int32).reshape(NW, N_CHUNKS, CHUNK)
  dst3 = edge_index[1].astype(jnp.int32).reshape(NW, N_CHUNKS, CHUNK)
  px = positions[:, 0]
  py = positions[:, 1]
  pz = positions[:, 2]

  l2c, srcc, dstc, counts = _k1(px, py, pz, src3, dst3)
  nf = _node_feats(node_attrs, W_embed)
  tpw = _tp_w(l2c, W_radial)
  srcc3 = srcc.reshape(NW, SEG // CHUNK, CHUNK)
  dstc3 = dstc.reshape(NW, SEG // CHUNK, CHUNK)
  aggp = _k3(nf, tpw, srcc3, dstc3, counts)[:, :N_NODES, :]

  batch2d = batch.astype(jnp.int32).reshape(N_NODES, 1)
  mmc2d = mm_charges.reshape(-1, 1)
  aew2d = atomic_energies_w.reshape(-1, 1)
  wf2d = W_field.reshape(1, HIDDEN)
  return _epilogue(aggp, nf, node_attrs, batch2d, positions,
                   mm_positions, mmc2d, aew2d, wf2d, W_readout)


# speedup vs baseline: 9.9747x; 9.9747x over previous
import functools

import jax
import jax.numpy as jnp
from jax import lax
from jax.experimental import pallas as pl
from jax.experimental.pallas import tpu as pltpu
from jax.experimental.pallas import tpu_sc as plsc

N_NODES = 10000
N_EDGES = 320000
HIDDEN = 128
NUM_BESSEL = 8
NUM_GRAPHS = 8
R_MAX = 5.0
R2_CUT = R_MAX * R_MAX
L2_SENTINEL = 4.0 * R2_CUT
P_CUTOFF = 5
AVG_NUM_NEIGHBORS = 32.0

NC = 2
NS = 16
NW = NC * NS
E_PER_W = N_EDGES // NW
CHUNK = 80
N_CHUNKS = E_PER_W // CHUNK
SEG = 10240
N_PAD = 10240
ROWS_PER_S = N_PAD // NS
EDGE_BLK = 2560
N_EDGE_BLKS = NW * SEG // EDGE_BLK
LAG = 8


def _k1_body(px_hbm, py_hbm, pz_hbm, src_hbm, dst_hbm,
             l2c_hbm, srcc_hbm, dstc_hbm, counts_hbm,
             sidx_v, didx_v, gbuf, sent_l2, sent_i, idxbuf, stage_l2, cbuf,
             spm_l2, spm_src, spm_dst, sem, sem2):
  sid = lax.axis_index("s")
  wid = lax.axis_index("c") * NS + sid
  base_o = wid * SEG
  base_s = sid * SEG

  pltpu.sync_copy(src_hbm.at[wid], sidx_v)
  pltpu.sync_copy(dst_hbm.at[wid], didx_v)

  def sfill(i, _):
    sl = pl.ds(i * 16, 16)
    sent_l2[sl] = jnp.full((16,), L2_SENTINEL, jnp.float32)
    sent_i[sl] = jnp.zeros((16,), jnp.int32)
    return 0
  lax.fori_loop(0, SEG // 16, sfill, 0)
  pltpu.sync_copy(sent_l2, spm_l2.at[pl.ds(base_s, SEG)])
  pltpu.sync_copy(sent_i, spm_src.at[pl.ds(base_s, SEG)])
  pltpu.sync_copy(sent_i, spm_dst.at[pl.ds(base_s, SEG)])

  tabs = (px_hbm, py_hbm, pz_hbm)

  def fire_g(k, b):
    for c in range(3):
      pltpu.async_copy(tabs[c].at[sidx_v.at[k]], gbuf.at[b, c], sem)
      pltpu.async_copy(tabs[c].at[didx_v.at[k]], gbuf.at[b, 3 + c], sem)

  def drain_g(k, b):
    for c in range(3):
      pltpu.make_async_copy(tabs[c].at[sidx_v.at[k]], gbuf.at[b, c],
                            sem).wait()
      pltpu.make_async_copy(tabs[c].at[didx_v.at[k]], gbuf.at[b, 3 + c],
                            sem).wait()

  def fire_s(k):
    pltpu.async_copy(stage_l2.at[k], spm_l2.at[idxbuf.at[k]], sem2)
    pltpu.async_copy(sidx_v.at[k], spm_src.at[idxbuf.at[k]], sem2)
    pltpu.async_copy(didx_v.at[k], spm_dst.at[idxbuf.at[k]], sem2)

  def drain_s(k):
    pltpu.make_async_copy(stage_l2.at[k], spm_l2.at[idxbuf.at[k]],
                          sem2).wait()
    pltpu.make_async_copy(sidx_v.at[k], spm_src.at[idxbuf.at[k]],
                          sem2).wait()
    pltpu.make_async_copy(didx_v.at[k], spm_dst.at[idxbuf.at[k]],
                          sem2).wait()

  def process(k, b, cnt):
    iota = lax.iota(jnp.int32, 16)
    for j in range(CHUNK // 16):
      sl = pl.ds(j * 16, 16)
      dx = gbuf[b, 3, sl] - gbuf[b, 0, sl]
      dy = gbuf[b, 4, sl] - gbuf[b, 1, sl]
      dz = gbuf[b, 5, sl] - gbuf[b, 2, sl]
      l2v = dx * dx + dy * dy + dz * dz
      mask = l2v < R2_CUT
      cs = jnp.where(mask, jnp.full((16,), 1, jnp.int32),
                     jnp.zeros((16,), jnp.int32))
      for d in (1, 2, 4, 8):
        sh = jnp.take(cs, jnp.maximum(iota - d, 0))
        cs = cs + jnp.where(iota >= d, sh, 0)
      idx = jnp.where(mask, base_s + cnt + cs - 1,
                      base_s + E_PER_W + j * 16 + iota)
      idxbuf[k, sl] = idx
      stage_l2[k, sl] = l2v
      cnt = cnt + cs[15]
    fire_s(k)

    @pl.when(k >= LAG)
    def _():
      drain_s(k - LAG)
    return cnt

  def chunk_body(k, cnt):
    cps = []
    for c in range(3):
      cps.append(pltpu.async_copy(tabs[c].at[sidx_v.at[k]],
                                  gbuf.at[0, c], sem))
      cps.append(pltpu.async_copy(tabs[c].at[didx_v.at[k]],
                                  gbuf.at[0, 3 + c], sem))
    for cp in cps:
      cp.wait()
    cnt = process(k, 0, cnt)
    return cnt

  cnt = lax.fori_loop(0, N_CHUNKS, chunk_body, jnp.int32(0))

  def tail_drain(k, _):
    drain_s(k)
    return 0
  lax.fori_loop(N_CHUNKS - LAG, N_CHUNKS, tail_drain, 0)

  pltpu.sync_copy(spm_l2.at[pl.ds(base_s, SEG)],
                  l2c_hbm.at[pl.ds(base_o, SEG)])
  pltpu.sync_copy(spm_src.at[pl.ds(base_s, SEG)],
                  srcc_hbm.at[pl.ds(base_o, SEG)])
  pltpu.sync_copy(spm_dst.at[pl.ds(base_s, SEG)],
                  dstc_hbm.at[pl.ds(base_o, SEG)])
  cbuf[...] = jnp.zeros((16,), jnp.int32) + cnt
  pltpu.sync_copy(cbuf, counts_hbm.at[wid])


_k1 = functools.partial(
    pl.kernel,
    out_type=(jax.ShapeDtypeStruct((NW * SEG,), jnp.float32),
              jax.ShapeDtypeStruct((NW * SEG,), jnp.int32),
              jax.ShapeDtypeStruct((NW * SEG,), jnp.int32),
              jax.ShapeDtypeStruct((NW, 16), jnp.int32)),
    mesh=plsc.VectorSubcoreMesh(core_axis_name="c", subcore_axis_name="s"),
    scratch_types=[
        pltpu.VMEM((N_CHUNKS, CHUNK), jnp.int32),
        pltpu.VMEM((N_CHUNKS, CHUNK), jnp.int32),
        pltpu.VMEM((2, 6, CHUNK), jnp.float32),
        pltpu.VMEM((SEG,), jnp.float32),
        pltpu.VMEM((SEG,), jnp.int32),
        pltpu.VMEM((N_CHUNKS, CHUNK), jnp.int32),
        pltpu.VMEM((N_CHUNKS, CHUNK), jnp.float32),
        pltpu.VMEM((16,), jnp.int32),
        pltpu.VMEM_SHARED((NS * SEG,), jnp.float32),
        pltpu.VMEM_SHARED((NS * SEG,), jnp.int32),
        pltpu.VMEM_SHARED((NS * SEG,), jnp.int32),
        pltpu.SemaphoreType.DMA,
        pltpu.SemaphoreType.DMA,
    ],
)(_k1_body)


def _k3_body(nf_hbm, tpw_hbm, srcc_hbm, dstc_hbm, counts_hbm, out_hbm,
             cidx_s, cidx_d, frows_v, tpw_v, cbuf, accum, sem):
  cid = lax.axis_index("c")
  sid = lax.axis_index("s")
  wid = cid * NS + sid
  base = wid * SEG

  pltpu.sync_copy(counts_hbm.at[wid], cbuf)
  cnt = cbuf[...][0]
  nch = (cnt + (CHUNK - 1)) // CHUNK

  def zrow(r, _):
    for cb in range(HIDDEN // 16):
      frows_v[0, r, pl.ds(cb * 16, 16)] = jnp.zeros((16,), jnp.float32)
    return 0
  lax.fori_loop(0, CHUNK, zrow, 0)
  for j in range(ROWS_PER_S // CHUNK):
    pltpu.sync_copy(frows_v.at[0],
                    accum.at[pl.ds(sid * ROWS_PER_S + j * CHUNK, CHUNK)])
  plsc.subcore_barrier()

  def fire(k, b):
    pltpu.sync_copy(srcc_hbm.at[wid, k], cidx_s.at[b])
    pltpu.sync_copy(dstc_hbm.at[wid, k], cidx_d.at[b])
    pltpu.async_copy(nf_hbm.at[cidx_s.at[b]], frows_v.at[b], sem)
    pltpu.async_copy(tpw_hbm.at[pl.ds(base + k * CHUNK, CHUNK)],
                     tpw_v.at[b], sem)

  def drain(k, b):
    pltpu.make_async_copy(nf_hbm.at[cidx_s.at[b]], frows_v.at[b],
                          sem).wait()
    pltpu.make_async_copy(tpw_hbm.at[pl.ds(base + k * CHUNK, CHUNK)],
                          tpw_v.at[b], sem).wait()

  def process(k, b):
    def mrow(r, _):
      for cb in range(HIDDEN // 16):
        sl = pl.ds(cb * 16, 16)
        frows_v[b, r, sl] = frows_v[b, r, sl] * tpw_v[b, r, sl]
      return 0
    lax.fori_loop(0, CHUNK, mrow, 0)
    pltpu.sync_copy(frows_v.at[b], accum.at[cidx_d.at[b]], add=True)

  @pl.when(nch > 0)
  def _():
    fire(0, 0)

  def pair_body(i, _):
    k0 = 2 * i

    @pl.when(k0 + 1 < nch)
    def _():
      fire(k0 + 1, 1)
    drain(k0, 0)
    process(k0, 0)

    @pl.when(k0 + 2 < nch)
    def _():
      fire(k0 + 2, 0)

    @pl.when(k0 + 1 < nch)
    def _():
      drain(k0 + 1, 1)
      process(k0 + 1, 1)
    return 0

  lax.fori_loop(0, (nch + 1) // 2, pair_body, 0)
  plsc.subcore_barrier()
  pltpu.sync_copy(accum.at[pl.ds(sid * ROWS_PER_S, ROWS_PER_S)],
                  out_hbm.at[cid, pl.ds(sid * ROWS_PER_S, ROWS_PER_S)])


_k3 = functools.partial(
    pl.kernel,
    out_type=jax.ShapeDtypeStruct((NC, N_PAD, HIDDEN), jnp.float32),
    mesh=plsc.VectorSubcoreMesh(core_axis_name="c", subcore_axis_name="s"),
    scratch_types=[
        pltpu.VMEM((2, CHUNK), jnp.int32),
        pltpu.VMEM((2, CHUNK), jnp.int32),
        pltpu.VMEM((2, CHUNK, HIDDEN), jnp.float32),
        pltpu.VMEM((2, CHUNK, HIDDEN), jnp.float32),
        pltpu.VMEM((16,), jnp.int32),
        pltpu.VMEM_SHARED((N_PAD, HIDDEN), jnp.float32),
        pltpu.SemaphoreType.DMA,
    ],
)(_k3_body)


def _k2a_body(na_ref, we_ref, out_ref):
  out_ref[...] = jnp.dot(na_ref[...], we_ref[...],
                         preferred_element_type=jnp.float32)


def _node_feats(node_attrs, w_embed):
  return pl.pallas_call(
      _k2a_body,
      out_shape=jax.ShapeDtypeStruct((N_NODES, HIDDEN), jnp.float32),
  )(node_attrs, w_embed)


def _k2b_body(l2_ref, wr_ref, out_ref):
  l2 = l2_ref[0, 0, :]
  lengths = jnp.sqrt(l2)
  r = jnp.maximum(lengths, 1e-6)
  n = (lax.broadcasted_iota(jnp.int32, (NUM_BESSEL, 1), 0) + 1
       ).astype(jnp.float32)
  bessel = (jnp.sqrt(2.0 / R_MAX)
            * jnp.sin(n * (jnp.pi / R_MAX) * r[None, :]) / r[None, :])
  x = lengths / R_MAX
  p = float(P_CUTOFF)
  xp = x ** p
  env = (1.0
         - ((p + 1.0) * (p + 2.0) / 2.0) * xp
         + p * (p + 2.0) * xp * x
         - (p * (p + 1.0) / 2.0) * xp * x * x)
  env = env * (x < 1.0).astype(jnp.float32)
  ef = bessel * env[None, :]
  out_ref[...] = lax.dot_general(
      ef, wr_ref[...],
      dimension_numbers=(((0,), (0,)), ((), ())),
      preferred_element_type=jnp.float32)


def _tp_w(l2, w_radial):
  l2_3d = l2.reshape(N_EDGE_BLKS, 1, EDGE_BLK)
  return pl.pallas_call(
      _k2b_body,
      grid=(N_EDGE_BLKS,),
      in_specs=[
          pl.BlockSpec((1, 1, EDGE_BLK), lambda i: (i, 0, 0)),
          pl.BlockSpec((NUM_BESSEL, HIDDEN), lambda i: (0, 0)),
      ],
      out_specs=pl.BlockSpec((EDGE_BLK, HIDDEN), lambda i: (i, 0)),
      out_shape=jax.ShapeDtypeStruct((NW * SEG, HIDDEN), jnp.float32),
  )(l2_3d, w_radial)


def _k4_body(aggp_ref, nf_ref, na_ref, batch_ref, pos_ref, mmp_ref, mmc_ref,
             aew_ref, wf_ref, wro_ref, out_ref):
  agg0 = (aggp_ref[0] + aggp_ref[1]) * (1.0 / AVG_NUM_NEIGHBORS)
  nf = nf_ref[...]
  dipole = lax.dot_general(mmc_ref[...], mmp_ref[...],
                           dimension_numbers=(((0,), (0,)), ((), ())),
                           preferred_element_type=jnp.float32)
  field_scal = lax.dot_general(pos_ref[...], dipole,
                               dimension_numbers=(((1,), (1,)), ((), ())),
                               preferred_element_type=jnp.float32)
  h = agg0 + nf + field_scal * wf_ref[...]
  h = h * jax.nn.sigmoid(h)
  ne = jnp.dot(h, wro_ref[...], preferred_element_type=jnp.float32)
  ne0 = jnp.dot(na_ref[...], aew_ref[...],
                preferred_element_type=jnp.float32)
  cat = jnp.concatenate([ne, ne0], axis=1)
  gids = lax.broadcasted_iota(jnp.int32, (N_NODES, NUM_GRAPHS), 1)
  m = (batch_ref[...] == gids).astype(jnp.float32)
  eng = lax.dot_general(m, cat,
                        dimension_numbers=(((0,), (0,)), ((), ())),
                        preferred_element_type=jnp.float32)
  out_ref[...] = eng[:, :3] + eng[:, 3:4]


def _epilogue(aggp, nf, node_attrs, batch2d, positions, mm_positions,
              mmc2d, aew2d, wf2d, w_readout):
  return pl.pallas_call(
      _k4_body,
      out_shape=jax.ShapeDtypeStruct((NUM_GRAPHS, 3), jnp.float32),
  )(aggp, nf, node_attrs, batch2d, positions, mm_positions, mmc2d,
    aew2d, wf2d, w_readout)


def kernel(positions, node_attrs, edge_index, shifts, batch, ptr,
           mm_positions, mm_charges, atomic_energies_w, W_embed,
           W_radial, W_field, W_readout):
  del ptr
  del shifts
  src3 = edge_index[0].astype(jnp.int32).reshape(NW, N_CHUNKS, CHUNK)
  dst3 = edge_index[1].astype(jnp.int32).reshape(NW, N_CHUNKS, CHUNK)
  px = positions[:, 0]
  py = positions[:, 1]
  pz = positions[:, 2]

  l2c, srcc, dstc, counts = _k1(px, py, pz, src3, dst3)
  nf = _node_feats(node_attrs, W_embed)
  tpw = _tp_w(l2c, W_radial)
  srcc3 = srcc.reshape(NW, SEG // CHUNK, CHUNK)
  dstc3 = dstc.reshape(NW, SEG // CHUNK, CHUNK)
  aggp = _k3(nf, tpw, srcc3, dstc3, counts)[:, :N_NODES, :]

  batch2d = batch.astype(jnp.int32).reshape(N_NODES, 1)
  mmc2d = mm_charges.reshape(-1, 1)
  aew2d = atomic_energies_w.reshape(-1, 1)
  wf2d = W_field.reshape(1, HIDDEN)
  return _epilogue(aggp, nf, node_attrs, batch2d, positions,
                   mm_positions, mmc2d, aew2d, wf2d, W_readout)


# speedup vs baseline: 10.9783x; 1.1006x over previous
import functools

import jax
import jax.numpy as jnp
from jax import lax
from jax.experimental import pallas as pl
from jax.experimental.pallas import tpu as pltpu
from jax.experimental.pallas import tpu_sc as plsc

N_NODES = 10000
N_EDGES = 320000
HIDDEN = 128
NUM_BESSEL = 8
NUM_GRAPHS = 8
R_MAX = 5.0
R2_CUT = R_MAX * R_MAX
L2_SENTINEL = 4.0 * R2_CUT
P_CUTOFF = 5
AVG_NUM_NEIGHBORS = 32.0

NC = 2
NS = 16
NW = NC * NS
E_PER_W = N_EDGES // NW
CHUNK = 80
N_CHUNKS = E_PER_W // CHUNK
SEG = 10240
N_PAD = 10240
ROWS_PER_S = N_PAD // NS
EDGE_BLK = 2560
N_EDGE_BLKS = NW * SEG // EDGE_BLK
LAG = 8


def _k1_body(px_hbm, py_hbm, pz_hbm, src_hbm, dst_hbm,
             l2c_hbm, srcc_hbm, dstc_hbm, counts_hbm,
             sidx_v, didx_v, gbuf, sent_l2, sent_i, idxbuf, stage_l2, cbuf,
             spm_l2, spm_src, spm_dst, sem, sem2):
  sid = lax.axis_index("s")
  wid = lax.axis_index("c") * NS + sid
  base_o = wid * SEG
  base_s = sid * SEG

  pltpu.sync_copy(src_hbm.at[wid], sidx_v)
  pltpu.sync_copy(dst_hbm.at[wid], didx_v)

  def sfill(i, _):
    sl = pl.ds(i * 16, 16)
    sent_l2[sl] = jnp.full((16,), L2_SENTINEL, jnp.float32)
    sent_i[sl] = jnp.zeros((16,), jnp.int32)
    return 0
  lax.fori_loop(0, SEG // 16, sfill, 0)
  pltpu.sync_copy(sent_l2, spm_l2.at[pl.ds(base_s, SEG)])
  pltpu.sync_copy(sent_i, spm_src.at[pl.ds(base_s, SEG)])
  pltpu.sync_copy(sent_i, spm_dst.at[pl.ds(base_s, SEG)])

  tabs = (px_hbm, py_hbm, pz_hbm)

  def fire_g(k, b):
    for c in range(3):
      pltpu.async_copy(tabs[c].at[sidx_v.at[k]], gbuf.at[b, c], sem)
      pltpu.async_copy(tabs[c].at[didx_v.at[k]], gbuf.at[b, 3 + c], sem)

  def drain_g(k, b):
    for c in range(3):
      pltpu.make_async_copy(tabs[c].at[sidx_v.at[k]], gbuf.at[b, c],
                            sem).wait()
      pltpu.make_async_copy(tabs[c].at[didx_v.at[k]], gbuf.at[b, 3 + c],
                            sem).wait()

  def fire_s(k):
    pltpu.async_copy(stage_l2.at[k], spm_l2.at[idxbuf.at[k]], sem2)
    pltpu.async_copy(sidx_v.at[k], spm_src.at[idxbuf.at[k]], sem2)
    pltpu.async_copy(didx_v.at[k], spm_dst.at[idxbuf.at[k]], sem2)

  def drain_s(k):
    pltpu.make_async_copy(stage_l2.at[k], spm_l2.at[idxbuf.at[k]],
                          sem2).wait()
    pltpu.make_async_copy(sidx_v.at[k], spm_src.at[idxbuf.at[k]],
                          sem2).wait()
    pltpu.make_async_copy(didx_v.at[k], spm_dst.at[idxbuf.at[k]],
                          sem2).wait()

  def process(k, b, cnt):
    iota = lax.iota(jnp.int32, 16)
    for j in range(CHUNK // 16):
      sl = pl.ds(j * 16, 16)
      dx = gbuf[b, 3, sl] - gbuf[b, 0, sl]
      dy = gbuf[b, 4, sl] - gbuf[b, 1, sl]
      dz = gbuf[b, 5, sl] - gbuf[b, 2, sl]
      l2v = dx * dx + dy * dy + dz * dz
      mask = l2v < R2_CUT
      cs = jnp.where(mask, jnp.full((16,), 1, jnp.int32),
                     jnp.zeros((16,), jnp.int32))
      for d in (1, 2, 4, 8):
        sh = jnp.take(cs, jnp.maximum(iota - d, 0))
        cs = cs + jnp.where(iota >= d, sh, 0)
      idx = jnp.where(mask, base_s + cnt + cs - 1,
                      base_s + E_PER_W + j * 16 + iota)
      idxbuf[k, sl] = idx
      stage_l2[k, sl] = l2v
      cnt = cnt + cs[15]
    fire_s(k)

    @pl.when(k >= LAG)
    def _():
      drain_s(k - LAG)
    return cnt

  fire_g(0, 0)

  def pair_body(i, cnt):
    k0 = 2 * i
    fire_g(k0 + 1, 1)
    drain_g(k0, 0)
    cnt = process(k0, 0, cnt)
    fire_g(k0 + 2, 0)
    drain_g(k0 + 1, 1)
    cnt = process(k0 + 1, 1, cnt)
    return cnt

  cnt = lax.fori_loop(0, (N_CHUNKS - 1) // 2, pair_body, jnp.int32(0))
  drain_g(N_CHUNKS - 1, 0)
  cnt = process(N_CHUNKS - 1, 0, cnt)

  def tail_drain(k, _):
    drain_s(k)
    return 0
  lax.fori_loop(N_CHUNKS - LAG, N_CHUNKS, tail_drain, 0)

  pltpu.sync_copy(spm_l2.at[pl.ds(base_s, SEG)],
                  l2c_hbm.at[pl.ds(base_o, SEG)])
  pltpu.sync_copy(spm_src.at[pl.ds(base_s, SEG)],
                  srcc_hbm.at[pl.ds(base_o, SEG)])
  pltpu.sync_copy(spm_dst.at[pl.ds(base_s, SEG)],
                  dstc_hbm.at[pl.ds(base_o, SEG)])
  cbuf[...] = jnp.zeros((16,), jnp.int32) + cnt
  pltpu.sync_copy(cbuf, counts_hbm.at[wid])


_k1 = functools.partial(
    pl.kernel,
    out_type=(jax.ShapeDtypeStruct((NW * SEG,), jnp.float32),
              jax.ShapeDtypeStruct((NW * SEG,), jnp.int32),
              jax.ShapeDtypeStruct((NW * SEG,), jnp.int32),
              jax.ShapeDtypeStruct((NW, 16), jnp.int32)),
    mesh=plsc.VectorSubcoreMesh(core_axis_name="c", subcore_axis_name="s"),
    scratch_types=[
        pltpu.VMEM((N_CHUNKS, CHUNK), jnp.int32),
        pltpu.VMEM((N_CHUNKS, CHUNK), jnp.int32),
        pltpu.VMEM((2, 6, CHUNK), jnp.float32),
        pltpu.VMEM((SEG,), jnp.float32),
        pltpu.VMEM((SEG,), jnp.int32),
        pltpu.VMEM((N_CHUNKS, CHUNK), jnp.int32),
        pltpu.VMEM((N_CHUNKS, CHUNK), jnp.float32),
        pltpu.VMEM((16,), jnp.int32),
        pltpu.VMEM_SHARED((NS * SEG,), jnp.float32),
        pltpu.VMEM_SHARED((NS * SEG,), jnp.int32),
        pltpu.VMEM_SHARED((NS * SEG,), jnp.int32),
        pltpu.SemaphoreType.DMA,
        pltpu.SemaphoreType.DMA,
    ],
)(_k1_body)


def _k3_body(nf_hbm, tpw_hbm, srcc_hbm, dstc_hbm, counts_hbm, out_hbm,
             cidx_s, cidx_d, frows_v, tpw_v, cbuf, accum, sem):
  cid = lax.axis_index("c")
  sid = lax.axis_index("s")
  wid = cid * NS + sid
  base = wid * SEG

  pltpu.sync_copy(counts_hbm.at[wid], cbuf)
  cnt = cbuf[...][0]
  nch = (cnt + (CHUNK - 1)) // CHUNK

  def zrow(r, _):
    for cb in range(HIDDEN // 16):
      frows_v[0, r, pl.ds(cb * 16, 16)] = jnp.zeros((16,), jnp.float32)
    return 0
  lax.fori_loop(0, CHUNK, zrow, 0)
  for j in range(ROWS_PER_S // CHUNK):
    pltpu.sync_copy(frows_v.at[0],
                    accum.at[pl.ds(sid * ROWS_PER_S + j * CHUNK, CHUNK)])
  plsc.subcore_barrier()

  def fire(k, b):
    pltpu.sync_copy(srcc_hbm.at[wid, k], cidx_s.at[b])
    pltpu.sync_copy(dstc_hbm.at[wid, k], cidx_d.at[b])
    pltpu.async_copy(nf_hbm.at[cidx_s.at[b]], frows_v.at[b], sem)
    pltpu.async_copy(tpw_hbm.at[pl.ds(base + k * CHUNK, CHUNK)],
                     tpw_v.at[b], sem)

  def drain(k, b):
    pltpu.make_async_copy(nf_hbm.at[cidx_s.at[b]], frows_v.at[b],
                          sem).wait()
    pltpu.make_async_copy(tpw_hbm.at[pl.ds(base + k * CHUNK, CHUNK)],
                          tpw_v.at[b], sem).wait()

  def process(k, b):
    def mrow(r, _):
      for cb in range(HIDDEN // 16):
        sl = pl.ds(cb * 16, 16)
        frows_v[b, r, sl] = frows_v[b, r, sl] * tpw_v[b, r, sl]
      return 0
    lax.fori_loop(0, CHUNK, mrow, 0)
    pltpu.sync_copy(frows_v.at[b], accum.at[cidx_d.at[b]], add=True)

  @pl.when(nch > 0)
  def _():
    fire(0, 0)

  def pair_body(i, _):
    k0 = 2 * i

    @pl.when(k0 + 1 < nch)
    def _():
      fire(k0 + 1, 1)
    drain(k0, 0)
    process(k0, 0)

    @pl.when(k0 + 2 < nch)
    def _():
      fire(k0 + 2, 0)

    @pl.when(k0 + 1 < nch)
    def _():
      drain(k0 + 1, 1)
      process(k0 + 1, 1)
    return 0

  lax.fori_loop(0, (nch + 1) // 2, pair_body, 0)
  plsc.subcore_barrier()
  pltpu.sync_copy(accum.at[pl.ds(sid * ROWS_PER_S, ROWS_PER_S)],
                  out_hbm.at[cid, pl.ds(sid * ROWS_PER_S, ROWS_PER_S)])


_k3 = functools.partial(
    pl.kernel,
    out_type=jax.ShapeDtypeStruct((NC, N_PAD, HIDDEN), jnp.float32),
    mesh=plsc.VectorSubcoreMesh(core_axis_name="c", subcore_axis_name="s"),
    scratch_types=[
        pltpu.VMEM((2, CHUNK), jnp.int32),
        pltpu.VMEM((2, CHUNK), jnp.int32),
        pltpu.VMEM((2, CHUNK, HIDDEN), jnp.float32),
        pltpu.VMEM((2, CHUNK, HIDDEN), jnp.float32),
        pltpu.VMEM((16,), jnp.int32),
        pltpu.VMEM_SHARED((N_PAD, HIDDEN), jnp.float32),
        pltpu.SemaphoreType.DMA,
    ],
)(_k3_body)


def _k2a_body(na_ref, we_ref, out_ref):
  out_ref[...] = jnp.dot(na_ref[...], we_ref[...],
                         preferred_element_type=jnp.float32)


def _node_feats(node_attrs, w_embed):
  return pl.pallas_call(
      _k2a_body,
      out_shape=jax.ShapeDtypeStruct((N_NODES, HIDDEN), jnp.float32),
  )(node_attrs, w_embed)


def _k2b_body(l2_ref, wr_ref, out_ref):
  l2 = l2_ref[0, 0, :]
  lengths = jnp.sqrt(l2)
  r = jnp.maximum(lengths, 1e-6)
  n = (lax.broadcasted_iota(jnp.int32, (NUM_BESSEL, 1), 0) + 1
       ).astype(jnp.float32)
  bessel = (jnp.sqrt(2.0 / R_MAX)
            * jnp.sin(n * (jnp.pi / R_MAX) * r[None, :]) / r[None, :])
  x = lengths / R_MAX
  p = float(P_CUTOFF)
  xp = x ** p
  env = (1.0
         - ((p + 1.0) * (p + 2.0) / 2.0) * xp
         + p * (p + 2.0) * xp * x
         - (p * (p + 1.0) / 2.0) * xp * x * x)
  env = env * (x < 1.0).astype(jnp.float32)
  ef = bessel * env[None, :]
  out_ref[...] = lax.dot_general(
      ef, wr_ref[...],
      dimension_numbers=(((0,), (0,)), ((), ())),
      preferred_element_type=jnp.float32)


def _tp_w(l2, w_radial):
  l2_3d = l2.reshape(N_EDGE_BLKS, 1, EDGE_BLK)
  return pl.pallas_call(
      _k2b_body,
      grid=(N_EDGE_BLKS,),
      in_specs=[
          pl.BlockSpec((1, 1, EDGE_BLK), lambda i: (i, 0, 0)),
          pl.BlockSpec((NUM_BESSEL, HIDDEN), lambda i: (0, 0)),
      ],
      out_specs=pl.BlockSpec((EDGE_BLK, HIDDEN), lambda i: (i, 0)),
      out_shape=jax.ShapeDtypeStruct((NW * SEG, HIDDEN), jnp.float32),
  )(l2_3d, w_radial)


def _k4_body(aggp_ref, nf_ref, na_ref, batch_ref, pos_ref, mmp_ref, mmc_ref,
             aew_ref, wf_ref, wro_ref, out_ref):
  agg0 = (aggp_ref[0] + aggp_ref[1]) * (1.0 / AVG_NUM_NEIGHBORS)
  nf = nf_ref[...]
  dipole = lax.dot_general(mmc_ref[...], mmp_ref[...],
                           dimension_numbers=(((0,), (0,)), ((), ())),
                           preferred_element_type=jnp.float32)
  field_scal = lax.dot_general(pos_ref[...], dipole,
                               dimension_numbers=(((1,), (1,)), ((), ())),
                               preferred_element_type=jnp.float32)
  h = agg0 + nf + field_scal * wf_ref[...]
  h = h * jax.nn.sigmoid(h)
  ne = jnp.dot(h, wro_ref[...], preferred_element_type=jnp.float32)
  ne0 = jnp.dot(na_ref[...], aew_ref[...],
                preferred_element_type=jnp.float32)
  cat = jnp.concatenate([ne, ne0], axis=1)
  gids = lax.broadcasted_iota(jnp.int32, (N_NODES, NUM_GRAPHS), 1)
  m = (batch_ref[...] == gids).astype(jnp.float32)
  eng = lax.dot_general(m, cat,
                        dimension_numbers=(((0,), (0,)), ((), ())),
                        preferred_element_type=jnp.float32)
  out_ref[...] = eng[:, :3] + eng[:, 3:4]


def _epilogue(aggp, nf, node_attrs, batch2d, positions, mm_positions,
              mmc2d, aew2d, wf2d, w_readout):
  return pl.pallas_call(
      _k4_body,
      out_shape=jax.ShapeDtypeStruct((NUM_GRAPHS, 3), jnp.float32),
  )(aggp, nf, node_attrs, batch2d, positions, mm_positions, mmc2d,
    aew2d, wf2d, w_readout)


def kernel(positions, node_attrs, edge_index, shifts, batch, ptr,
           mm_positions, mm_charges, atomic_energies_w, W_embed,
           W_radial, W_field, W_readout):
  del ptr
  del shifts
  src3 = edge_index[0].astype(jnp.int32).reshape(NW, N_CHUNKS, CHUNK)
  dst3 = edge_index[1].astype(jnp.int32).reshape(NW, N_CHUNKS, CHUNK)
  px = positions[:, 0]
  py = positions[:, 1]
  pz = positions[:, 2]

  l2c, srcc, dstc, counts = _k1(px, py, pz, src3, dst3)
  nf = _node_feats(node_attrs, W_embed)
  tpw = _tp_w(l2c, W_radial)
  srcc3 = srcc.reshape(NW, SEG // CHUNK, CHUNK)
  dstc3 = dstc.reshape(NW, SEG // CHUNK, CHUNK)
  aggp = _k3(nf, tpw, srcc3, dstc3, counts)[:, :N_NODES, :]

  batch2d = batch.astype(jnp.int32).reshape(N_NODES, 1)
  mmc2d = mm_charges.reshape(-1, 1)
  aew2d = atomic_energies_w.reshape(-1, 1)
  wf2d = W_field.reshape(1, HIDDEN)
  return _epilogue(aggp, nf, node_attrs, batch2d, positions,
                   mm_positions, mmc2d, aew2d, wf2d, W_readout)


# speedup vs baseline: 11.0560x; 1.0071x over previous
import functools

import jax
import jax.numpy as jnp
from jax import lax
from jax.experimental import pallas as pl
from jax.experimental.pallas import tpu as pltpu
from jax.experimental.pallas import tpu_sc as plsc

N_NODES = 10000
N_EDGES = 320000
HIDDEN = 128
NUM_BESSEL = 8
NUM_GRAPHS = 8
R_MAX = 5.0
R2_CUT = R_MAX * R_MAX
L2_SENTINEL = 4.0 * R2_CUT
P_CUTOFF = 5
AVG_NUM_NEIGHBORS = 32.0

NC = 2
NS = 16
NW = NC * NS
E_PER_W = N_EDGES // NW
CHUNK = 80
N_CHUNKS = E_PER_W // CHUNK
SEG = 10240
N_PAD = 10240
ROWS_PER_S = N_PAD // NS
EDGE_BLK = 2560
N_EDGE_BLKS = NW * SEG // EDGE_BLK
LAG = 8


def _k1_body(px_hbm, py_hbm, pz_hbm, src_hbm, dst_hbm,
             l2c_hbm, srcc_hbm, dstc_hbm, counts_hbm,
             sidx_v, didx_v, gbuf, sent_l2, sent_i, idxbuf, stage_l2, cbuf,
             spm_l2, spm_src, spm_dst, sem, semb, sem2):
  sid = lax.axis_index("s")
  wid = lax.axis_index("c") * NS + sid
  base_o = wid * SEG
  base_s = sid * SEG

  pltpu.sync_copy(src_hbm.at[wid], sidx_v)
  pltpu.sync_copy(dst_hbm.at[wid], didx_v)

  def sfill(i, _):
    sl = pl.ds(i * 16, 16)
    sent_l2[sl] = jnp.full((16,), L2_SENTINEL, jnp.float32)
    sent_i[sl] = jnp.zeros((16,), jnp.int32)
    return 0
  lax.fori_loop(0, SEG // 16, sfill, 0)
  pltpu.sync_copy(sent_l2, spm_l2.at[pl.ds(base_s, SEG)])
  pltpu.sync_copy(sent_i, spm_src.at[pl.ds(base_s, SEG)])
  pltpu.sync_copy(sent_i, spm_dst.at[pl.ds(base_s, SEG)])

  tabs = (px_hbm, py_hbm, pz_hbm)

  gsems = (sem, semb)

  def fire_g(k, b):
    for c in range(3):
      pltpu.async_copy(tabs[c].at[sidx_v.at[k]], gbuf.at[b, c], gsems[b])
      pltpu.async_copy(tabs[c].at[didx_v.at[k]], gbuf.at[b, 3 + c],
                       gsems[b])

  def drain_g(k, b):
    for c in range(3):
      pltpu.make_async_copy(tabs[c].at[sidx_v.at[k]], gbuf.at[b, c],
                            gsems[b]).wait()
      pltpu.make_async_copy(tabs[c].at[didx_v.at[k]], gbuf.at[b, 3 + c],
                            gsems[b]).wait()

  def fire_s(k):
    pltpu.async_copy(stage_l2.at[k], spm_l2.at[idxbuf.at[k]], sem2)
    pltpu.async_copy(sidx_v.at[k], spm_src.at[idxbuf.at[k]], sem2)
    pltpu.async_copy(didx_v.at[k], spm_dst.at[idxbuf.at[k]], sem2)

  def drain_s(k):
    pltpu.make_async_copy(stage_l2.at[k], spm_l2.at[idxbuf.at[k]],
                          sem2).wait()
    pltpu.make_async_copy(sidx_v.at[k], spm_src.at[idxbuf.at[k]],
                          sem2).wait()
    pltpu.make_async_copy(didx_v.at[k], spm_dst.at[idxbuf.at[k]],
                          sem2).wait()

  def process(k, b, cnt):
    iota = lax.iota(jnp.int32, 16)
    for j in range(CHUNK // 16):
      sl = pl.ds(j * 16, 16)
      dx = gbuf[b, 3, sl] - gbuf[b, 0, sl]
      dy = gbuf[b, 4, sl] - gbuf[b, 1, sl]
      dz = gbuf[b, 5, sl] - gbuf[b, 2, sl]
      l2v = dx * dx + dy * dy + dz * dz
      mask = l2v < R2_CUT
      cs = jnp.where(mask, jnp.full((16,), 1, jnp.int32),
                     jnp.zeros((16,), jnp.int32))
      for d in (1, 2, 4, 8):
        sh = jnp.take(cs, jnp.maximum(iota - d, 0))
        cs = cs + jnp.where(iota >= d, sh, 0)
      idx = jnp.where(mask, base_s + cnt + cs - 1,
                      base_s + E_PER_W + j * 16 + iota)
      idxbuf[k, sl] = idx
      stage_l2[k, sl] = l2v
      cnt = cnt + cs[15]
    fire_s(k)

    @pl.when(k >= LAG)
    def _():
      drain_s(k - LAG)
    return cnt

  fire_g(0, 0)

  def pair_body(i, cnt):
    k0 = 2 * i
    fire_g(k0 + 1, 1)
    drain_g(k0, 0)
    cnt = process(k0, 0, cnt)
    fire_g(k0 + 2, 0)
    drain_g(k0 + 1, 1)
    cnt = process(k0 + 1, 1, cnt)
    return cnt

  cnt = lax.fori_loop(0, (N_CHUNKS - 1) // 2, pair_body, jnp.int32(0))
  drain_g(N_CHUNKS - 1, 0)
  cnt = process(N_CHUNKS - 1, 0, cnt)

  def tail_drain(k, _):
    drain_s(k)
    return 0
  lax.fori_loop(N_CHUNKS - LAG, N_CHUNKS, tail_drain, 0)

  pltpu.sync_copy(spm_l2.at[pl.ds(base_s, SEG)],
                  l2c_hbm.at[pl.ds(base_o, SEG)])
  pltpu.sync_copy(spm_src.at[pl.ds(base_s, SEG)],
                  srcc_hbm.at[pl.ds(base_o, SEG)])
  pltpu.sync_copy(spm_dst.at[pl.ds(base_s, SEG)],
                  dstc_hbm.at[pl.ds(base_o, SEG)])
  cbuf[...] = jnp.zeros((16,), jnp.int32) + cnt
  pltpu.sync_copy(cbuf, counts_hbm.at[wid])


_k1 = functools.partial(
    pl.kernel,
    out_type=(jax.ShapeDtypeStruct((NW * SEG,), jnp.float32),
              jax.ShapeDtypeStruct((NW * SEG,), jnp.int32),
              jax.ShapeDtypeStruct((NW * SEG,), jnp.int32),
              jax.ShapeDtypeStruct((NW, 16), jnp.int32)),
    mesh=plsc.VectorSubcoreMesh(core_axis_name="c", subcore_axis_name="s"),
    scratch_types=[
        pltpu.VMEM((N_CHUNKS, CHUNK), jnp.int32),
        pltpu.VMEM((N_CHUNKS, CHUNK), jnp.int32),
        pltpu.VMEM((2, 6, CHUNK), jnp.float32),
        pltpu.VMEM((SEG,), jnp.float32),
        pltpu.VMEM((SEG,), jnp.int32),
        pltpu.VMEM((N_CHUNKS, CHUNK), jnp.int32),
        pltpu.VMEM((N_CHUNKS, CHUNK), jnp.float32),
        pltpu.VMEM((16,), jnp.int32),
        pltpu.VMEM_SHARED((NS * SEG,), jnp.float32),
        pltpu.VMEM_SHARED((NS * SEG,), jnp.int32),
        pltpu.VMEM_SHARED((NS * SEG,), jnp.int32),
        pltpu.SemaphoreType.DMA,
        pltpu.SemaphoreType.DMA,
        pltpu.SemaphoreType.DMA,
    ],
)(_k1_body)


def _k3_body(nf_hbm, tpw_hbm, srcc_hbm, dstc_hbm, counts_hbm, out_hbm,
             cidx_s, cidx_d, frows_v, tpw_v, cbuf, accum, sem, semb):
  cid = lax.axis_index("c")
  sid = lax.axis_index("s")
  wid = cid * NS + sid
  base = wid * SEG

  pltpu.sync_copy(counts_hbm.at[wid], cbuf)
  cnt = cbuf[...][0]
  nch = (cnt + (CHUNK - 1)) // CHUNK

  def zrow(r, _):
    for cb in range(HIDDEN // 16):
      frows_v[0, r, pl.ds(cb * 16, 16)] = jnp.zeros((16,), jnp.float32)
    return 0
  lax.fori_loop(0, CHUNK, zrow, 0)
  for j in range(ROWS_PER_S // CHUNK):
    pltpu.sync_copy(frows_v.at[0],
                    accum.at[pl.ds(sid * ROWS_PER_S + j * CHUNK, CHUNK)])
  plsc.subcore_barrier()

  ksems = (sem, semb)

  def fire(k, b):
    pltpu.sync_copy(srcc_hbm.at[wid, k], cidx_s.at[b])
    pltpu.sync_copy(dstc_hbm.at[wid, k], cidx_d.at[b])
    pltpu.async_copy(nf_hbm.at[cidx_s.at[b]], frows_v.at[b], ksems[b])
    pltpu.async_copy(tpw_hbm.at[pl.ds(base + k * CHUNK, CHUNK)],
                     tpw_v.at[b], ksems[b])

  def drain(k, b):
    pltpu.make_async_copy(nf_hbm.at[cidx_s.at[b]], frows_v.at[b],
                          ksems[b]).wait()
    pltpu.make_async_copy(tpw_hbm.at[pl.ds(base + k * CHUNK, CHUNK)],
                          tpw_v.at[b], ksems[b]).wait()

  def process(k, b):
    def mrow(r, _):
      for cb in range(HIDDEN // 16):
        sl = pl.ds(cb * 16, 16)
        frows_v[b, r, sl] = frows_v[b, r, sl] * tpw_v[b, r, sl]
      return 0
    lax.fori_loop(0, CHUNK, mrow, 0)
    pltpu.sync_copy(frows_v.at[b], accum.at[cidx_d.at[b]], add=True)

  @pl.when(nch > 0)
  def _():
    fire(0, 0)

  def pair_body(i, _):
    k0 = 2 * i

    @pl.when(k0 + 1 < nch)
    def _():
      fire(k0 + 1, 1)
    drain(k0, 0)
    process(k0, 0)

    @pl.when(k0 + 2 < nch)
    def _():
      fire(k0 + 2, 0)

    @pl.when(k0 + 1 < nch)
    def _():
      drain(k0 + 1, 1)
      process(k0 + 1, 1)
    return 0

  lax.fori_loop(0, (nch + 1) // 2, pair_body, 0)
  plsc.subcore_barrier()
  pltpu.sync_copy(accum.at[pl.ds(sid * ROWS_PER_S, ROWS_PER_S)],
                  out_hbm.at[cid, pl.ds(sid * ROWS_PER_S, ROWS_PER_S)])


_k3 = functools.partial(
    pl.kernel,
    out_type=jax.ShapeDtypeStruct((NC, N_PAD, HIDDEN), jnp.float32),
    mesh=plsc.VectorSubcoreMesh(core_axis_name="c", subcore_axis_name="s"),
    scratch_types=[
        pltpu.VMEM((2, CHUNK), jnp.int32),
        pltpu.VMEM((2, CHUNK), jnp.int32),
        pltpu.VMEM((2, CHUNK, HIDDEN), jnp.float32),
        pltpu.VMEM((2, CHUNK, HIDDEN), jnp.float32),
        pltpu.VMEM((16,), jnp.int32),
        pltpu.VMEM_SHARED((N_PAD, HIDDEN), jnp.float32),
        pltpu.SemaphoreType.DMA,
        pltpu.SemaphoreType.DMA,
    ],
)(_k3_body)


def _k2a_body(na_ref, we_ref, out_ref):
  out_ref[...] = jnp.dot(na_ref[...], we_ref[...],
                         preferred_element_type=jnp.float32)


def _node_feats(node_attrs, w_embed):
  return pl.pallas_call(
      _k2a_body,
      out_shape=jax.ShapeDtypeStruct((N_NODES, HIDDEN), jnp.float32),
  )(node_attrs, w_embed)


def _k2b_body(l2_ref, wr_ref, out_ref):
  l2 = l2_ref[0, 0, :]
  lengths = jnp.sqrt(l2)
  r = jnp.maximum(lengths, 1e-6)
  n = (lax.broadcasted_iota(jnp.int32, (NUM_BESSEL, 1), 0) + 1
       ).astype(jnp.float32)
  bessel = (jnp.sqrt(2.0 / R_MAX)
            * jnp.sin(n * (jnp.pi / R_MAX) * r[None, :]) / r[None, :])
  x = lengths / R_MAX
  p = float(P_CUTOFF)
  xp = x ** p
  env = (1.0
         - ((p + 1.0) * (p + 2.0) / 2.0) * xp
         + p * (p + 2.0) * xp * x
         - (p * (p + 1.0) / 2.0) * xp * x * x)
  env = env * (x < 1.0).astype(jnp.float32)
  ef = bessel * env[None, :]
  out_ref[...] = lax.dot_general(
      ef, wr_ref[...],
      dimension_numbers=(((0,), (0,)), ((), ())),
      preferred_element_type=jnp.float32)


def _tp_w(l2, w_radial):
  l2_3d = l2.reshape(N_EDGE_BLKS, 1, EDGE_BLK)
  return pl.pallas_call(
      _k2b_body,
      grid=(N_EDGE_BLKS,),
      in_specs=[
          pl.BlockSpec((1, 1, EDGE_BLK), lambda i: (i, 0, 0)),
          pl.BlockSpec((NUM_BESSEL, HIDDEN), lambda i: (0, 0)),
      ],
      out_specs=pl.BlockSpec((EDGE_BLK, HIDDEN), lambda i: (i, 0)),
      out_shape=jax.ShapeDtypeStruct((NW * SEG, HIDDEN), jnp.float32),
  )(l2_3d, w_radial)


def _k4_body(aggp_ref, nf_ref, na_ref, batch_ref, pos_ref, mmp_ref, mmc_ref,
             aew_ref, wf_ref, wro_ref, out_ref):
  agg0 = (aggp_ref[0] + aggp_ref[1]) * (1.0 / AVG_NUM_NEIGHBORS)
  nf = nf_ref[...]
  dipole = lax.dot_general(mmc_ref[...], mmp_ref[...],
                           dimension_numbers=(((0,), (0,)), ((), ())),
                           preferred_element_type=jnp.float32)
  field_scal = lax.dot_general(pos_ref[...], dipole,
                               dimension_numbers=(((1,), (1,)), ((), ())),
                               preferred_element_type=jnp.float32)
  h = agg0 + nf + field_scal * wf_ref[...]
  h = h * jax.nn.sigmoid(h)
  ne = jnp.dot(h, wro_ref[...], preferred_element_type=jnp.float32)
  ne0 = jnp.dot(na_ref[...], aew_ref[...],
                preferred_element_type=jnp.float32)
  cat = jnp.concatenate([ne, ne0], axis=1)
  gids = lax.broadcasted_iota(jnp.int32, (N_NODES, NUM_GRAPHS), 1)
  m = (batch_ref[...] == gids).astype(jnp.float32)
  eng = lax.dot_general(m, cat,
                        dimension_numbers=(((0,), (0,)), ((), ())),
                        preferred_element_type=jnp.float32)
  out_ref[...] = eng[:, :3] + eng[:, 3:4]


def _epilogue(aggp, nf, node_attrs, batch2d, positions, mm_positions,
              mmc2d, aew2d, wf2d, w_readout):
  return pl.pallas_call(
      _k4_body,
      out_shape=jax.ShapeDtypeStruct((NUM_GRAPHS, 3), jnp.float32),
  )(aggp, nf, node_attrs, batch2d, positions, mm_positions, mmc2d,
    aew2d, wf2d, w_readout)


def kernel(positions, node_attrs, edge_index, shifts, batch, ptr,
           mm_positions, mm_charges, atomic_energies_w, W_embed,
           W_radial, W_field, W_readout):
  del ptr
  del shifts
  src3 = edge_index[0].astype(jnp.int32).reshape(NW, N_CHUNKS, CHUNK)
  dst3 = edge_index[1].astype(jnp.int32).reshape(NW, N_CHUNKS, CHUNK)
  px = positions[:, 0]
  py = positions[:, 1]
  pz = positions[:, 2]

  l2c, srcc, dstc, counts = _k1(px, py, pz, src3, dst3)
  nf = _node_feats(node_attrs, W_embed)
  tpw = _tp_w(l2c, W_radial)
  srcc3 = srcc.reshape(NW, SEG // CHUNK, CHUNK)
  dstc3 = dstc.reshape(NW, SEG // CHUNK, CHUNK)
  aggp = _k3(nf, tpw, srcc3, dstc3, counts)[:, :N_NODES, :]

  batch2d = batch.astype(jnp.int32).reshape(N_NODES, 1)
  mmc2d = mm_charges.reshape(-1, 1)
  aew2d = atomic_energies_w.reshape(-1, 1)
  wf2d = W_field.reshape(1, HIDDEN)
  return _epilogue(aggp, nf, node_attrs, batch2d, positions,
                   mm_positions, mmc2d, aew2d, wf2d, W_readout)


# speedup vs baseline: 11.5474x; 1.0444x over previous
import functools

import jax
import jax.numpy as jnp
from jax import lax
from jax.experimental import pallas as pl
from jax.experimental.pallas import tpu as pltpu
from jax.experimental.pallas import tpu_sc as plsc

N_NODES = 10000
N_EDGES = 320000
HIDDEN = 128
NUM_BESSEL = 8
NUM_GRAPHS = 8
R_MAX = 5.0
R2_CUT = R_MAX * R_MAX
L2_SENTINEL = 4.0 * R2_CUT
P_CUTOFF = 5
AVG_NUM_NEIGHBORS = 32.0

NC = 2
NS = 16
NW = NC * NS
E_PER_W = N_EDGES // NW
CHUNK = 80
N_CHUNKS = E_PER_W // CHUNK
SEG = 10240
N_PAD = 10240
ROWS_PER_S = N_PAD // NS
EDGE_BLK = 2560
N_EDGE_BLKS = NW * SEG // EDGE_BLK
LAG = 8


def _k1_body(px_hbm, py_hbm, pz_hbm, src_hbm, dst_hbm,
             l2c_hbm, srcc_hbm, dstc_hbm, counts_hbm,
             sidx_v, didx_v, gbuf, sent_l2, sent_i, idxbuf, stage_l2, cbuf,
             spm_l2, spm_src, spm_dst, sem, semb, sem2):
  sid = lax.axis_index("s")
  wid = lax.axis_index("c") * NS + sid
  base_o = wid * SEG
  base_s = sid * SEG

  pltpu.sync_copy(src_hbm.at[wid], sidx_v)
  pltpu.sync_copy(dst_hbm.at[wid], didx_v)

  def sfill(i, _):
    sl = pl.ds(i * 16, 16)
    sent_l2[sl] = jnp.full((16,), L2_SENTINEL, jnp.float32)
    sent_i[sl] = jnp.zeros((16,), jnp.int32)
    return 0
  lax.fori_loop(0, SEG // 16, sfill, 0)
  pltpu.sync_copy(sent_l2, spm_l2.at[pl.ds(base_s, SEG)])
  pltpu.sync_copy(sent_i, spm_src.at[pl.ds(base_s, SEG)])
  pltpu.sync_copy(sent_i, spm_dst.at[pl.ds(base_s, SEG)])

  tabs = (px_hbm, py_hbm, pz_hbm)

  gsems = (sem, semb)

  def fire_g(k, b):
    for c in range(3):
      pltpu.async_copy(tabs[c].at[sidx_v.at[k]], gbuf.at[b, c], gsems[b])
      pltpu.async_copy(tabs[c].at[didx_v.at[k]], gbuf.at[b, 3 + c],
                       gsems[b])

  def drain_g(k, b):
    for c in range(3):
      pltpu.make_async_copy(tabs[c].at[sidx_v.at[k]], gbuf.at[b, c],
                            gsems[b]).wait()
      pltpu.make_async_copy(tabs[c].at[didx_v.at[k]], gbuf.at[b, 3 + c],
                            gsems[b]).wait()

  def fire_s(k):
    pltpu.async_copy(stage_l2.at[k], spm_l2.at[idxbuf.at[k]], sem2)
    pltpu.async_copy(sidx_v.at[k], spm_src.at[idxbuf.at[k]], sem2)
    pltpu.async_copy(didx_v.at[k], spm_dst.at[idxbuf.at[k]], sem2)

  def drain_s(k):
    pltpu.make_async_copy(stage_l2.at[k], spm_l2.at[idxbuf.at[k]],
                          sem2).wait()
    pltpu.make_async_copy(sidx_v.at[k], spm_src.at[idxbuf.at[k]],
                          sem2).wait()
    pltpu.make_async_copy(didx_v.at[k], spm_dst.at[idxbuf.at[k]],
                          sem2).wait()

  def process(k, b, cnt):
    iota = lax.iota(jnp.int32, 16)
    for j in range(CHUNK // 16):
      sl = pl.ds(j * 16, 16)
      dx = gbuf[b, 3, sl] - gbuf[b, 0, sl]
      dy = gbuf[b, 4, sl] - gbuf[b, 1, sl]
      dz = gbuf[b, 5, sl] - gbuf[b, 2, sl]
      l2v = dx * dx + dy * dy + dz * dz
      mask = l2v < R2_CUT
      cs = jnp.where(mask, jnp.full((16,), 1, jnp.int32),
                     jnp.zeros((16,), jnp.int32))
      for d in (1, 2, 4, 8):
        sh = jnp.take(cs, jnp.maximum(iota - d, 0))
        cs = cs + jnp.where(iota >= d, sh, 0)
      idx = jnp.where(mask, base_s + cnt + cs - 1,
                      base_s + E_PER_W + j * 16 + iota)
      idxbuf[k, sl] = idx
      stage_l2[k, sl] = l2v
      cnt = cnt + cs[15]
    fire_s(k)

    @pl.when(k >= LAG)
    def _():
      drain_s(k - LAG)
    return cnt

  fire_g(0, 0)

  def pair_body(i, cnt):
    k0 = 2 * i
    fire_g(k0 + 1, 1)
    drain_g(k0, 0)
    cnt = process(k0, 0, cnt)
    fire_g(k0 + 2, 0)
    drain_g(k0 + 1, 1)
    cnt = process(k0 + 1, 1, cnt)
    return cnt

  cnt = lax.fori_loop(0, (N_CHUNKS - 1) // 2, pair_body, jnp.int32(0))
  drain_g(N_CHUNKS - 1, 0)
  cnt = process(N_CHUNKS - 1, 0, cnt)

  def tail_drain(k, _):
    drain_s(k)
    return 0
  lax.fori_loop(N_CHUNKS - LAG, N_CHUNKS, tail_drain, 0)

  pltpu.sync_copy(spm_l2.at[pl.ds(base_s, SEG)],
                  l2c_hbm.at[pl.ds(base_o, SEG)])
  pltpu.sync_copy(spm_src.at[pl.ds(base_s, SEG)],
                  srcc_hbm.at[pl.ds(base_o, SEG)])
  pltpu.sync_copy(spm_dst.at[pl.ds(base_s, SEG)],
                  dstc_hbm.at[pl.ds(base_o, SEG)])
  cbuf[...] = jnp.zeros((16,), jnp.int32) + cnt
  pltpu.sync_copy(cbuf, counts_hbm.at[wid])


_k1 = functools.partial(
    pl.kernel,
    out_type=(jax.ShapeDtypeStruct((NW * SEG,), jnp.float32),
              jax.ShapeDtypeStruct((NW * SEG,), jnp.int32),
              jax.ShapeDtypeStruct((NW * SEG,), jnp.int32),
              jax.ShapeDtypeStruct((NW, 16), jnp.int32)),
    mesh=plsc.VectorSubcoreMesh(core_axis_name="c", subcore_axis_name="s"),
    scratch_types=[
        pltpu.VMEM((N_CHUNKS, CHUNK), jnp.int32),
        pltpu.VMEM((N_CHUNKS, CHUNK), jnp.int32),
        pltpu.VMEM((2, 6, CHUNK), jnp.float32),
        pltpu.VMEM((SEG,), jnp.float32),
        pltpu.VMEM((SEG,), jnp.int32),
        pltpu.VMEM((N_CHUNKS, CHUNK), jnp.int32),
        pltpu.VMEM((N_CHUNKS, CHUNK), jnp.float32),
        pltpu.VMEM((16,), jnp.int32),
        pltpu.VMEM_SHARED((NS * SEG,), jnp.float32),
        pltpu.VMEM_SHARED((NS * SEG,), jnp.int32),
        pltpu.VMEM_SHARED((NS * SEG,), jnp.int32),
        pltpu.SemaphoreType.DMA,
        pltpu.SemaphoreType.DMA,
        pltpu.SemaphoreType.DMA,
    ],
)(_k1_body)


def _k3_body(nf_hbm, tpw_hbm, srcc_hbm, dstc_hbm, counts_hbm, out_hbm,
             cidx_s, cidx_d, frows_v, tpw_v, cbuf, accum, sem, semb):
  cid = lax.axis_index("c")
  sid = lax.axis_index("s")
  wid = cid * NS + sid
  base = wid * SEG

  pltpu.sync_copy(counts_hbm.at[wid], cbuf)
  cnt = cbuf[...][0]
  nch = (cnt + (CHUNK - 1)) // CHUNK

  def zrow(r, _):
    for cb in range(HIDDEN // 16):
      frows_v[0, r, pl.ds(cb * 16, 16)] = jnp.zeros((16,), jnp.float32)
    return 0
  lax.fori_loop(0, CHUNK, zrow, 0)
  for j in range(ROWS_PER_S // CHUNK):
    pltpu.sync_copy(frows_v.at[0],
                    accum.at[pl.ds(sid * ROWS_PER_S + j * CHUNK, CHUNK)])
  plsc.subcore_barrier()

  ksems = (sem, semb)

  def fire(k, b):
    pltpu.sync_copy(srcc_hbm.at[wid, k], cidx_s.at[b])
    pltpu.sync_copy(dstc_hbm.at[wid, k], cidx_d.at[b])
    pltpu.async_copy(nf_hbm.at[cidx_s.at[b]], frows_v.at[b], ksems[b])
    pltpu.async_copy(tpw_hbm.at[pl.ds(base + k * CHUNK, CHUNK)],
                     tpw_v.at[b], ksems[b])

  def drain(k, b):
    pltpu.make_async_copy(nf_hbm.at[cidx_s.at[b]], frows_v.at[b],
                          ksems[b]).wait()
    pltpu.make_async_copy(tpw_hbm.at[pl.ds(base + k * CHUNK, CHUNK)],
                          tpw_v.at[b], ksems[b]).wait()

  def process(k, b):
    @functools.partial(plsc.parallel_loop, 0, CHUNK, unroll=4)
    def _(r):
      for cb in range(HIDDEN // 16):
        sl = pl.ds(cb * 16, 16)
        frows_v[b, r, sl] = frows_v[b, r, sl] * tpw_v[b, r, sl]
    pltpu.sync_copy(frows_v.at[b], accum.at[cidx_d.at[b]], add=True)

  @pl.when(nch > 0)
  def _():
    fire(0, 0)

  def pair_body(i, _):
    k0 = 2 * i

    @pl.when(k0 + 1 < nch)
    def _():
      fire(k0 + 1, 1)
    drain(k0, 0)
    process(k0, 0)

    @pl.when(k0 + 2 < nch)
    def _():
      fire(k0 + 2, 0)

    @pl.when(k0 + 1 < nch)
    def _():
      drain(k0 + 1, 1)
      process(k0 + 1, 1)
    return 0

  lax.fori_loop(0, (nch + 1) // 2, pair_body, 0)
  plsc.subcore_barrier()
  pltpu.sync_copy(accum.at[pl.ds(sid * ROWS_PER_S, ROWS_PER_S)],
                  out_hbm.at[cid, pl.ds(sid * ROWS_PER_S, ROWS_PER_S)])


_k3 = functools.partial(
    pl.kernel,
    out_type=jax.ShapeDtypeStruct((NC, N_PAD, HIDDEN), jnp.float32),
    mesh=plsc.VectorSubcoreMesh(core_axis_name="c", subcore_axis_name="s"),
    scratch_types=[
        pltpu.VMEM((2, CHUNK), jnp.int32),
        pltpu.VMEM((2, CHUNK), jnp.int32),
        pltpu.VMEM((2, CHUNK, HIDDEN), jnp.float32),
        pltpu.VMEM((2, CHUNK, HIDDEN), jnp.float32),
        pltpu.VMEM((16,), jnp.int32),
        pltpu.VMEM_SHARED((N_PAD, HIDDEN), jnp.float32),
        pltpu.SemaphoreType.DMA,
        pltpu.SemaphoreType.DMA,
    ],
)(_k3_body)


def _k2a_body(na_ref, we_ref, out_ref):
  out_ref[...] = jnp.dot(na_ref[...], we_ref[...],
                         preferred_element_type=jnp.float32)


def _node_feats(node_attrs, w_embed):
  return pl.pallas_call(
      _k2a_body,
      out_shape=jax.ShapeDtypeStruct((N_NODES, HIDDEN), jnp.float32),
  )(node_attrs, w_embed)


def _k2b_body(l2_ref, wr_ref, out_ref):
  l2 = l2_ref[0, 0, :]
  lengths = jnp.sqrt(l2)
  r = jnp.maximum(lengths, 1e-6)
  n = (lax.broadcasted_iota(jnp.int32, (NUM_BESSEL, 1), 0) + 1
       ).astype(jnp.float32)
  bessel = (jnp.sqrt(2.0 / R_MAX)
            * jnp.sin(n * (jnp.pi / R_MAX) * r[None, :]) / r[None, :])
  x = lengths / R_MAX
  p = float(P_CUTOFF)
  xp = x ** p
  env = (1.0
         - ((p + 1.0) * (p + 2.0) / 2.0) * xp
         + p * (p + 2.0) * xp * x
         - (p * (p + 1.0) / 2.0) * xp * x * x)
  env = env * (x < 1.0).astype(jnp.float32)
  ef = bessel * env[None, :]
  out_ref[...] = lax.dot_general(
      ef, wr_ref[...],
      dimension_numbers=(((0,), (0,)), ((), ())),
      preferred_element_type=jnp.float32)


def _tp_w(l2, w_radial):
  l2_3d = l2.reshape(N_EDGE_BLKS, 1, EDGE_BLK)
  return pl.pallas_call(
      _k2b_body,
      grid=(N_EDGE_BLKS,),
      in_specs=[
          pl.BlockSpec((1, 1, EDGE_BLK), lambda i: (i, 0, 0)),
          pl.BlockSpec((NUM_BESSEL, HIDDEN), lambda i: (0, 0)),
      ],
      out_specs=pl.BlockSpec((EDGE_BLK, HIDDEN), lambda i: (i, 0)),
      out_shape=jax.ShapeDtypeStruct((NW * SEG, HIDDEN), jnp.float32),
  )(l2_3d, w_radial)


def _k4_body(aggp_ref, nf_ref, na_ref, batch_ref, pos_ref, mmp_ref, mmc_ref,
             aew_ref, wf_ref, wro_ref, out_ref):
  agg0 = (aggp_ref[0] + aggp_ref[1]) * (1.0 / AVG_NUM_NEIGHBORS)
  nf = nf_ref[...]
  dipole = lax.dot_general(mmc_ref[...], mmp_ref[...],
                           dimension_numbers=(((0,), (0,)), ((), ())),
                           preferred_element_type=jnp.float32)
  field_scal = lax.dot_general(pos_ref[...], dipole,
                               dimension_numbers=(((1,), (1,)), ((), ())),
                               preferred_element_type=jnp.float32)
  h = agg0 + nf + field_scal * wf_ref[...]
  h = h * jax.nn.sigmoid(h)
  ne = jnp.dot(h, wro_ref[...], preferred_element_type=jnp.float32)
  ne0 = jnp.dot(na_ref[...], aew_ref[...],
                preferred_element_type=jnp.float32)
  cat = jnp.concatenate([ne, ne0], axis=1)
  gids = lax.broadcasted_iota(jnp.int32, (N_NODES, NUM_GRAPHS), 1)
  m = (batch_ref[...] == gids).astype(jnp.float32)
  eng = lax.dot_general(m, cat,
                        dimension_numbers=(((0,), (0,)), ((), ())),
                        preferred_element_type=jnp.float32)
  out_ref[...] = eng[:, :3] + eng[:, 3:4]


def _epilogue(aggp, nf, node_attrs, batch2d, positions, mm_positions,
              mmc2d, aew2d, wf2d, w_readout):
  return pl.pallas_call(
      _k4_body,
      out_shape=jax.ShapeDtypeStruct((NUM_GRAPHS, 3), jnp.float32),
  )(aggp, nf, node_attrs, batch2d, positions, mm_positions, mmc2d,
    aew2d, wf2d, w_readout)


def kernel(positions, node_attrs, edge_index, shifts, batch, ptr,
           mm_positions, mm_charges, atomic_energies_w, W_embed,
           W_radial, W_field, W_readout):
  del ptr
  del shifts
  src3 = edge_index[0].astype(jnp.int32).reshape(NW, N_CHUNKS, CHUNK)
  dst3 = edge_index[1].astype(jnp.int32).reshape(NW, N_CHUNKS, CHUNK)
  px = positions[:, 0]
  py = positions[:, 1]
  pz = positions[:, 2]

  l2c, srcc, dstc, counts = _k1(px, py, pz, src3, dst3)
  nf = _node_feats(node_attrs, W_embed)
  tpw = _tp_w(l2c, W_radial)
  srcc3 = srcc.reshape(NW, SEG // CHUNK, CHUNK)
  dstc3 = dstc.reshape(NW, SEG // CHUNK, CHUNK)
  aggp = _k3(nf, tpw, srcc3, dstc3, counts)[:, :N_NODES, :]

  batch2d = batch.astype(jnp.int32).reshape(N_NODES, 1)
  mmc2d = mm_charges.reshape(-1, 1)
  aew2d = atomic_energies_w.reshape(-1, 1)
  wf2d = W_field.reshape(1, HIDDEN)
  return _epilogue(aggp, nf, node_attrs, batch2d, positions,
                   mm_positions, mmc2d, aew2d, wf2d, W_readout)
